# baseline (speedup 1.0000x reference)
import functools
import math

import jax
import jax.numpy as jnp
from jax import lax
from jax.experimental import pallas as pl
from jax.experimental.pallas import tpu as pltpu
from jax.experimental.pallas import tpu_sc as plsc

D_MODEL = 1024
RANK = 64
N_COMPRESS = 16
N_KNOWLEDGE = 32768
TOPK = 8
B_SZ = 2
S_LEN = 2048

TS = 256
KT = 2048
N_KTILES = N_KNOWLEDGE // KT

GRP = 128
NGRP = N_KNOWLEDGE // GRP
GRP_PER_KT = KT // GRP

NTOK = B_SZ * S_LEN
NW = 32
TSC2 = 256
TC4 = 256
NCAND = TOPK * GRP

_INT_MAX = 2147483647
_NEG_INF = float("-inf")


def _sc_body(mw_ref, cnf_ref, out_ref):
    out_ref[...] = jnp.dot(mw_ref[...], cnf_ref[...],
                           preferred_element_type=jnp.float32)


def _shared_compress(memory_weights, cn_flat):
    return pl.pallas_call(
        _sc_body,
        out_shape=jax.ShapeDtypeStruct((B_SZ, D_MODEL * RANK), jnp.float32),
    )(memory_weights, cn_flat)


def _q_body(x_ref, sc_ref, out_ref):
    out_ref[0] = jnp.dot(x_ref[0], sc_ref[0],
                         preferred_element_type=jnp.float32)


def _q_proj(x, sc):
    return pl.pallas_call(
        _q_body,
        grid=(B_SZ,),
        in_specs=[
            pl.BlockSpec((1, S_LEN, D_MODEL), lambda b: (b, 0, 0)),
            pl.BlockSpec((1, D_MODEL, RANK), lambda b: (b, 0, 0)),
        ],
        out_specs=pl.BlockSpec((1, S_LEN, RANK), lambda b: (b, 0, 0)),
        out_shape=jax.ShapeDtypeStruct((B_SZ, S_LEN, RANK), jnp.float32),
    )(x, sc)


def _extract8(vals, idxs):
    tv, ti = [], []
    work = vals
    for _ in range(TOPK):
        m = jnp.max(work, axis=1, keepdims=True)
        ism = work == m
        am = jnp.min(jnp.where(ism, idxs, _INT_MAX), axis=1, keepdims=True)
        tv.append(m)
        ti.append(am)
        work = jnp.where(idxs == am, _NEG_INF, work)
    return jnp.concatenate(tv, axis=1), jnp.concatenate(ti, axis=1)


def _c1_body(q_ref, k_ref, s_ref, m_ref):
    q = q_ref[0]
    kk = k_ref[...]
    s = lax.dot_general(q, kk, (((1,), (1,)), ((), ())),
                        preferred_element_type=jnp.float32)
    s = s * (1.0 / math.sqrt(RANK))
    st = s.reshape(TS // 8, 8, GRP_PER_KT, GRP).transpose(0, 2, 1, 3)
    s_ref[0] = st
    gm = [jnp.max(s[:, i * GRP:(i + 1) * GRP], axis=1, keepdims=True)
          for i in range(GRP_PER_KT)]
    m_ref[0, 0] = jnp.concatenate(gm, axis=1)


def _c1(q, knowledge_K):
    return pl.pallas_call(
        _c1_body,
        grid=(B_SZ, S_LEN // TS, N_KTILES),
        in_specs=[
            pl.BlockSpec((1, TS, RANK), lambda b, s, k: (b, s, 0)),
            pl.BlockSpec((KT, RANK), lambda b, s, k: (k, 0)),
        ],
        out_specs=[
            pl.BlockSpec((1, TS // 8, GRP_PER_KT, 8, GRP),
                         lambda b, s, k: (b, s, k, 0, 0)),
            pl.BlockSpec((1, 1, TS, GRP_PER_KT), lambda b, s, k: (b, k, s, 0)),
        ],
        out_shape=[
            jax.ShapeDtypeStruct((B_SZ, S_LEN // 8, NGRP, 8, GRP),
                                 jnp.float32),
            jax.ShapeDtypeStruct((B_SZ, N_KTILES, S_LEN, GRP_PER_KT),
                                 jnp.float32),
        ],
        compiler_params=pltpu.CompilerParams(
            dimension_semantics=("parallel", "parallel", "parallel")),
    )(q, knowledge_K)


def _c2_body(m_ref, g_ref):
    m = m_ref[0]
    gid = lax.broadcasted_iota(jnp.int32, (TSC2, NGRP), 1)
    _, ti = _extract8(m, gid)
    g_ref[0] = ti


def _c2(gmax):
    return pl.pallas_call(
        _c2_body,
        grid=(B_SZ, S_LEN // TSC2),
        in_specs=[pl.BlockSpec((1, TSC2, NGRP), lambda b, s: (b, s, 0))],
        out_specs=pl.BlockSpec((1, TSC2, TOPK), lambda b, s: (b, s, 0)),
        out_shape=jax.ShapeDtypeStruct((B_SZ, S_LEN, TOPK), jnp.int32),
    )(gmax)


def _make_sc_gather(n_rows_out, row_w, rows_per_dma, table_rows):
    idx_per_w = n_rows_out // NW
    n_chunks = idx_per_w // rows_per_dma

    def body(t_hbm, idx_hbm, out_hbm, idx_v, rows_v, sem):
        wid = lax.axis_index("s") * 2 + lax.axis_index("c")
        base = wid * idx_per_w
        pltpu.sync_copy(idx_hbm.at[pl.ds(base, idx_per_w)], idx_v)
        for c in range(n_chunks):
            pltpu.async_copy(
                t_hbm.at[idx_v.at[pl.ds(c * rows_per_dma, rows_per_dma)]],
                rows_v, sem).wait()
            pltpu.sync_copy(
                rows_v,
                out_hbm.at[pl.ds(base + c * rows_per_dma, rows_per_dma)])

    return pl.kernel(
        body,
        mesh=plsc.VectorSubcoreMesh(core_axis_name="c", subcore_axis_name="s"),
        out_type=jax.ShapeDtypeStruct((n_rows_out, row_w), jnp.float32),
        scratch_types=[
            pltpu.VMEM((idx_per_w,), jnp.int32),
            pltpu.VMEM((rows_per_dma, row_w), jnp.float32),
            pltpu.SemaphoreType.DMA,
        ],
    )


@functools.lru_cache(maxsize=2)
def _sc_gather_scores():
    return _make_sc_gather(NTOK * TOPK, 8 * GRP, 64, NTOK * NGRP // 8)


@functools.lru_cache(maxsize=2)
def _sc_gather_v():
    return _make_sc_gather(NTOK * TOPK, D_MODEL, 64, N_KNOWLEDGE)


def _c4_body(cand_ref, gid_ref, iout_ref, wout_ref):
    gids = gid_ref[...]
    lane = lax.broadcasted_iota(jnp.int32, (TC4, GRP), 1)
    tokr = lax.broadcasted_iota(jnp.int32, (TC4, GRP), 0) % 8
    parts = []
    idx_parts = []
    for j in range(TOPK):
        acc = jnp.zeros((TC4, GRP), jnp.float32)
        for r in range(8):
            acc = jnp.where(tokr == r, cand_ref[:, j, r, :], acc)
        parts.append(acc)
        idx_parts.append(gids[:, j:j + 1] * GRP + lane)
    cand = jnp.concatenate(parts, axis=1)
    gidx = jnp.concatenate(idx_parts, axis=1)
    tv, ti = _extract8(cand, gidx)
    e = jnp.exp(tv - jnp.max(tv, axis=1, keepdims=True))
    wout_ref[...] = e / jnp.sum(e, axis=1, keepdims=True)
    iout_ref[...] = ti


def _c4(cand, gids):
    return pl.pallas_call(
        _c4_body,
        grid=(NTOK // TC4,),
        in_specs=[
            pl.BlockSpec((TC4, TOPK, 8, GRP), lambda t: (t, 0, 0, 0)),
            pl.BlockSpec((TC4, TOPK), lambda t: (t, 0)),
        ],
        out_specs=[
            pl.BlockSpec((TC4, TOPK), lambda t: (t, 0)),
            pl.BlockSpec((TC4, TOPK), lambda t: (t, 0)),
        ],
        out_shape=[
            jax.ShapeDtypeStruct((NTOK, TOPK), jnp.int32),
            jax.ShapeDtypeStruct((NTOK, TOPK), jnp.float32),
        ],
    )(cand, gids)


COMB_TS = 64


def _comb_body(sel_ref, w_ref, out_ref):
    sel = sel_ref[...]
    w = w_ref[...]
    out_ref[...] = jnp.sum(sel * w[..., None], axis=1)


def _combine(sel, w):
    return pl.pallas_call(
        _comb_body,
        grid=(NTOK // COMB_TS,),
        in_specs=[
            pl.BlockSpec((COMB_TS, TOPK, D_MODEL), lambda t: (t, 0, 0)),
            pl.BlockSpec((COMB_TS, TOPK), lambda t: (t, 0)),
        ],
        out_specs=pl.BlockSpec((COMB_TS, D_MODEL), lambda t: (t, 0)),
        out_shape=jax.ShapeDtypeStruct((NTOK, D_MODEL), jnp.float32),
    )(sel, w)


def kernel(x, memory_weights, compress_neurons, knowledge_K, knowledge_V):
    cn_flat = compress_neurons.reshape(N_COMPRESS, D_MODEL * RANK)
    sc = _shared_compress(memory_weights, cn_flat)
    sc = sc.reshape(B_SZ, D_MODEL, RANK)
    q = _q_proj(x, sc)

    scores, gmax4 = _c1(q, knowledge_K)
    gmax = gmax4.transpose(0, 2, 1, 3).reshape(B_SZ, S_LEN, NGRP)
    gids = _c2(gmax)

    gids_flat = gids.reshape(NTOK, TOPK)
    score_tiles = scores.reshape(NTOK * NGRP // 8, 8 * GRP)
    row_ids = ((jnp.arange(NTOK, dtype=jnp.int32) // 8 * NGRP)[:, None]
               + gids_flat)
    cand = _sc_gather_scores()(score_tiles, row_ids.reshape(-1))
    cand = cand.reshape(NTOK, TOPK, 8, GRP)

    topk_idx_flat, weights_flat = _c4(cand, gids_flat)

    sel = _sc_gather_v()(knowledge_V, topk_idx_flat.reshape(-1))
    sel = sel.reshape(NTOK, TOPK, D_MODEL)
    out = _combine(sel, weights_flat)

    return (out.reshape(B_SZ, S_LEN, D_MODEL),
            topk_idx_flat.reshape(B_SZ, S_LEN, TOPK),
            weights_flat.reshape(B_SZ, S_LEN, TOPK))


# speedup vs baseline: 1.0298x; 1.0298x over previous
import functools
import math

import jax
import jax.numpy as jnp
from jax import lax
from jax.experimental import pallas as pl
from jax.experimental.pallas import tpu as pltpu
from jax.experimental.pallas import tpu_sc as plsc

D_MODEL = 1024
RANK = 64
N_COMPRESS = 16
N_KNOWLEDGE = 32768
TOPK = 8
B_SZ = 2
S_LEN = 2048

TS = 256
KT = 2048
N_KTILES = N_KNOWLEDGE // KT

GRP = 128
NGRP = N_KNOWLEDGE // GRP
GRP_PER_KT = KT // GRP

NTOK = B_SZ * S_LEN
NW = 32
TSC2 = 256
TC4 = 256
NCAND = TOPK * GRP

_INT_MAX = 2147483647
_NEG_INF = float("-inf")


def _sc_body(mw_ref, cnf_ref, out_ref):
    out_ref[...] = jnp.dot(mw_ref[...], cnf_ref[...],
                           preferred_element_type=jnp.float32)


def _shared_compress(memory_weights, cn_flat):
    return pl.pallas_call(
        _sc_body,
        out_shape=jax.ShapeDtypeStruct((B_SZ, D_MODEL * RANK), jnp.float32),
    )(memory_weights, cn_flat)


def _q_body(x_ref, sc_ref, out_ref):
    out_ref[0] = jnp.dot(x_ref[0], sc_ref[0],
                         preferred_element_type=jnp.float32)


def _q_proj(x, sc):
    return pl.pallas_call(
        _q_body,
        grid=(B_SZ,),
        in_specs=[
            pl.BlockSpec((1, S_LEN, D_MODEL), lambda b: (b, 0, 0)),
            pl.BlockSpec((1, D_MODEL, RANK), lambda b: (b, 0, 0)),
        ],
        out_specs=pl.BlockSpec((1, S_LEN, RANK), lambda b: (b, 0, 0)),
        out_shape=jax.ShapeDtypeStruct((B_SZ, S_LEN, RANK), jnp.float32),
    )(x, sc)


def _extract8(vals, idxs):
    tv, ti = [], []
    work = vals
    for _ in range(TOPK):
        m = jnp.max(work, axis=1, keepdims=True)
        ism = work == m
        am = jnp.min(jnp.where(ism, idxs, _INT_MAX), axis=1, keepdims=True)
        tv.append(m)
        ti.append(am)
        work = jnp.where(idxs == am, _NEG_INF, work)
    return jnp.concatenate(tv, axis=1), jnp.concatenate(ti, axis=1)


def _c1_body(q_ref, k_ref, s_ref, m_ref):
    q = q_ref[0]
    kk = k_ref[...]
    s = lax.dot_general(q, kk, (((1,), (1,)), ((), ())),
                        preferred_element_type=jnp.float32)
    s = s * (1.0 / math.sqrt(RANK))
    s_ref[0] = s
    gm = [jnp.max(s[:, i * GRP:(i + 1) * GRP], axis=1, keepdims=True)
          for i in range(GRP_PER_KT)]
    m_ref[0, 0] = jnp.concatenate(gm, axis=1)


def _c1(q, knowledge_K):
    return pl.pallas_call(
        _c1_body,
        grid=(B_SZ, S_LEN // TS, N_KTILES),
        in_specs=[
            pl.BlockSpec((1, TS, RANK), lambda b, s, k: (b, s, 0)),
            pl.BlockSpec((KT, RANK), lambda b, s, k: (k, 0)),
        ],
        out_specs=[
            pl.BlockSpec((1, TS, KT), lambda b, s, k: (b, s, k)),
            pl.BlockSpec((1, 1, TS, GRP_PER_KT), lambda b, s, k: (b, k, s, 0)),
        ],
        out_shape=[
            jax.ShapeDtypeStruct((B_SZ, S_LEN, N_KNOWLEDGE), jnp.float32),
            jax.ShapeDtypeStruct((B_SZ, N_KTILES, S_LEN, GRP_PER_KT),
                                 jnp.float32),
        ],
        compiler_params=pltpu.CompilerParams(
            dimension_semantics=("parallel", "parallel", "parallel")),
    )(q, knowledge_K)


def _c2_body(m_ref, g_ref):
    m = m_ref[0]
    gid = lax.broadcasted_iota(jnp.int32, (TSC2, NGRP), 1)
    _, ti = _extract8(m, gid)
    g_ref[0] = ti


def _c2(gmax):
    return pl.pallas_call(
        _c2_body,
        grid=(B_SZ, S_LEN // TSC2),
        in_specs=[pl.BlockSpec((1, TSC2, NGRP), lambda b, s: (b, s, 0))],
        out_specs=pl.BlockSpec((1, TSC2, TOPK), lambda b, s: (b, s, 0)),
        out_shape=jax.ShapeDtypeStruct((B_SZ, S_LEN, TOPK), jnp.int32),
    )(gmax)


def _make_sc_gather(n_rows_out, row_w, rows_per_dma, table_rows):
    idx_per_w = n_rows_out // NW
    n_chunks = idx_per_w // rows_per_dma

    def body(t_hbm, idx_hbm, out_hbm, idx_v, rows0_v, rows1_v, sem0, sem1):
        wid = lax.axis_index("s") * 2 + lax.axis_index("c")
        base = wid * idx_per_w
        pltpu.sync_copy(idx_hbm.at[pl.ds(base, idx_per_w)], idx_v)
        bufs = (rows0_v, rows1_v)
        sems = (sem0, sem1)

        def start(c):
            return pltpu.async_copy(
                t_hbm.at[idx_v.at[pl.ds(c * rows_per_dma, rows_per_dma)]],
                bufs[c % 2], sems[c % 2])

        cp = start(0)
        for c in range(n_chunks):
            nxt = None
            if c + 1 < n_chunks:
                nxt = start(c + 1)
            cp.wait()
            pltpu.sync_copy(
                bufs[c % 2],
                out_hbm.at[pl.ds(base + c * rows_per_dma, rows_per_dma)])
            cp = nxt

    return pl.kernel(
        body,
        mesh=plsc.VectorSubcoreMesh(core_axis_name="c", subcore_axis_name="s"),
        out_type=jax.ShapeDtypeStruct((n_rows_out, row_w), jnp.float32),
        scratch_types=[
            pltpu.VMEM((idx_per_w,), jnp.int32),
            pltpu.VMEM((rows_per_dma, row_w), jnp.float32),
            pltpu.VMEM((rows_per_dma, row_w), jnp.float32),
            pltpu.SemaphoreType.DMA,
            pltpu.SemaphoreType.DMA,
        ],
    )


@functools.lru_cache(maxsize=2)
def _sc_gather_scores():
    return _make_sc_gather(NTOK * TOPK, 8 * GRP, 32, NTOK * NGRP // 8)


@functools.lru_cache(maxsize=2)
def _sc_gather_v():
    return _make_sc_gather(NTOK * TOPK, D_MODEL, 32, N_KNOWLEDGE)


def _c4_body(cand_ref, gid_ref, iout_ref, wout_ref):
    gids = gid_ref[...]
    lane = lax.broadcasted_iota(jnp.int32, (TC4, GRP), 1)
    tokr = lax.broadcasted_iota(jnp.int32, (TC4, GRP), 0) % 8
    parts = []
    idx_parts = []
    for j in range(TOPK):
        acc = jnp.zeros((TC4, GRP), jnp.float32)
        for r in range(8):
            acc = jnp.where(tokr == r, cand_ref[:, j, r, :], acc)
        parts.append(acc)
        idx_parts.append(gids[:, j:j + 1] * GRP + lane)
    cand = jnp.concatenate(parts, axis=1)
    gidx = jnp.concatenate(idx_parts, axis=1)
    tv, ti = _extract8(cand, gidx)
    e = jnp.exp(tv - jnp.max(tv, axis=1, keepdims=True))
    wout_ref[...] = e / jnp.sum(e, axis=1, keepdims=True)
    iout_ref[...] = ti


def _c4(cand, gids):
    return pl.pallas_call(
        _c4_body,
        grid=(NTOK // TC4,),
        in_specs=[
            pl.BlockSpec((TC4, TOPK, 8, GRP), lambda t: (t, 0, 0, 0)),
            pl.BlockSpec((TC4, TOPK), lambda t: (t, 0)),
        ],
        out_specs=[
            pl.BlockSpec((TC4, TOPK), lambda t: (t, 0)),
            pl.BlockSpec((TC4, TOPK), lambda t: (t, 0)),
        ],
        out_shape=[
            jax.ShapeDtypeStruct((NTOK, TOPK), jnp.int32),
            jax.ShapeDtypeStruct((NTOK, TOPK), jnp.float32),
        ],
    )(cand, gids)


COMB_TS = 64


def _comb_body(sel_ref, w_ref, out_ref):
    sel = sel_ref[...]
    w = w_ref[...]
    out_ref[...] = jnp.sum(sel * w[..., None], axis=1)


def _combine(sel, w):
    return pl.pallas_call(
        _comb_body,
        grid=(NTOK // COMB_TS,),
        in_specs=[
            pl.BlockSpec((COMB_TS, TOPK, D_MODEL), lambda t: (t, 0, 0)),
            pl.BlockSpec((COMB_TS, TOPK), lambda t: (t, 0)),
        ],
        out_specs=pl.BlockSpec((COMB_TS, D_MODEL), lambda t: (t, 0)),
        out_shape=jax.ShapeDtypeStruct((NTOK, D_MODEL), jnp.float32),
    )(sel, w)


def kernel(x, memory_weights, compress_neurons, knowledge_K, knowledge_V):
    cn_flat = compress_neurons.reshape(N_COMPRESS, D_MODEL * RANK)
    sc = _shared_compress(memory_weights, cn_flat)
    sc = sc.reshape(B_SZ, D_MODEL, RANK)
    q = _q_proj(x, sc)

    scores, gmax4 = _c1(q, knowledge_K)
    gmax = gmax4.transpose(0, 2, 1, 3).reshape(B_SZ, S_LEN, NGRP)
    gids = _c2(gmax)

    gids_flat = gids.reshape(NTOK, TOPK)
    score_tiles = (scores.reshape(B_SZ, S_LEN // 8, 8, NGRP, GRP)
                   .transpose(0, 1, 3, 2, 4)
                   .reshape(NTOK * NGRP // 8, 8 * GRP))
    row_ids = ((jnp.arange(NTOK, dtype=jnp.int32) // 8 * NGRP)[:, None]
               + gids_flat)
    cand = _sc_gather_scores()(score_tiles, row_ids.reshape(-1))
    cand = cand.reshape(NTOK, TOPK, 8, GRP)

    topk_idx_flat, weights_flat = _c4(cand, gids_flat)

    sel = _sc_gather_v()(knowledge_V, topk_idx_flat.reshape(-1))
    sel = sel.reshape(NTOK, TOPK, D_MODEL)
    out = _combine(sel, weights_flat)

    return (out.reshape(B_SZ, S_LEN, D_MODEL),
            topk_idx_flat.reshape(B_SZ, S_LEN, TOPK),
            weights_flat.reshape(B_SZ, S_LEN, TOPK))
